# baseline (speedup 1.0000x reference)
import functools

import jax
import jax.numpy as jnp
from jax import lax
from jax.experimental import pallas as pl
from jax.experimental.pallas import tpu as pltpu
from jax.experimental.pallas import tpu_sc as plsc

B, R, N, L, D, H, Dh = 4, 8, 2048, 256, 3072, 24, 128
BR = B * R
D2 = 2 * D

E_BLK = 512
O_BLK = 512
N_BLK = 256
NB = N // N_BLK
NC, NS = 2, 16
LANES = 16


@functools.partial(
    pl.kernel,
    out_type=jax.ShapeDtypeStruct((B * NB, N_BLK, R), jnp.int32),
    mesh=plsc.VectorSubcoreMesh(core_axis_name="c", subcore_axis_name="s"),
    compiler_params=pltpu.CompilerParams(needs_layout_passes=False),
    scratch_types=[
        pltpu.VMEM((R, L), jnp.int32),
        pltpu.VMEM((1, N_BLK), jnp.int32),
        pltpu.VMEM((N_BLK, R), jnp.int32),
        pltpu.VMEM((LANES,), jnp.float32),
    ],
)
def _winner_sc(idx_hbm, mask_hbm, win_hbm, idx_v, win_v, rep_v, mask_v):
    wid = lax.axis_index("s") * NC + lax.axis_index("c")
    b = wid // NB
    n0 = (wid % NB) * N_BLK
    pltpu.sync_copy(idx_hbm.at[b], idx_v)
    pltpu.sync_copy(mask_hbm.at[b], mask_v)
    mv = mask_v[...]
    minus1 = jnp.full((LANES,), -1, jnp.int32)
    zeros = jnp.zeros((LANES,), jnp.int32)
    lane = lax.iota(jnp.int32, LANES)
    for c in range(N_BLK // LANES):
        win_v[0, pl.ds(c * LANES, LANES)] = minus1
    for j in range(R):
        valid = mv[j] == 1.0
        jvec = jnp.full((LANES,), j, jnp.int32)
        for c in range(L // LANES):
            idx = idx_v[j, pl.ds(c * LANES, LANES)]
            rel = idx - n0
            inb = (rel >= 0) & (rel < N_BLK)
            inb = jnp.logical_and(inb, valid)
            relc = jnp.clip(rel, 0, N_BLK - 1)
            plsc.store_scatter(win_v, [zeros, relc], jvec, mask=inb)
    for c in range(N_BLK // LANES):
        wchunk = win_v[0, pl.ds(c * LANES, LANES)]
        rowidx = c * LANES + lane
        for col in range(R):
            colv = jnp.full((LANES,), col, jnp.int32)
            plsc.store_scatter(rep_v, [rowidx, colv], wchunk)
    pltpu.sync_copy(rep_v, win_hbm.at[wid])


def _layout_kv_body(x_ref, w_ref, b_ref, alpha_ref, mask_ref,
                    lo_ref, kvs_ref):
    kv = lax.dot_general(x_ref[...], w_ref[...], (((1,), (1,)), ((), ())),
                         preferred_element_type=jnp.float32)
    kv = kv + b_ref[...]
    valid = mask_ref[...] == 1.0
    lo_ref[...] = jnp.where(valid, kv, 0.0)
    kvs_ref[...] = kv * (1.0 - alpha_ref[...])


def _out_vec_body(v_ref, w_ref, o_ref):
    o_ref[...] = lax.dot_general(v_ref[...], w_ref[...],
                                 (((1,), (1,)), ((), ())),
                                 preferred_element_type=jnp.float32)


def _assemble_body(img_ref, win_ref, ov_ref, out_ref):
    w8 = win_ref[0]
    jot = lax.broadcasted_iota(jnp.int32, (N_BLK, R), 1)
    oh = (w8 == jot).astype(jnp.float32)
    rows = lax.dot_general(oh, ov_ref[0], (((1,), (0,)), ((), ())),
                           preferred_element_type=jnp.float32)
    out_ref[0] = jnp.where(w8[:, 0:1] >= 0, rows, img_ref[0])


def kernel(instance_tokens, image_tokens, img_idxs, layout_masks, alpha,
           W_lh, b_lh, W_q, W_out):
    x = instance_tokens.reshape(BR, D)
    alpha2 = alpha.reshape(BR, 1)
    mask2 = layout_masks.reshape(BR, 1)
    b_lh2 = b_lh.reshape(1, D2)

    mask16 = jnp.pad(layout_masks, ((0, 0), (0, LANES - R)))
    winner = _winner_sc(img_idxs, mask16)

    lo, kvs = pl.pallas_call(
        _layout_kv_body,
        grid=(D2 // E_BLK,),
        in_specs=[
            pl.BlockSpec((BR, D), lambda e: (0, 0)),
            pl.BlockSpec((E_BLK, D), lambda e: (e, 0)),
            pl.BlockSpec((1, E_BLK), lambda e: (0, e)),
            pl.BlockSpec((BR, 1), lambda e: (0, 0)),
            pl.BlockSpec((BR, 1), lambda e: (0, 0)),
        ],
        out_specs=[
            pl.BlockSpec((BR, E_BLK), lambda e: (0, e)),
            pl.BlockSpec((BR, E_BLK), lambda e: (0, e)),
        ],
        out_shape=[
            jax.ShapeDtypeStruct((BR, D2), jnp.float32),
            jax.ShapeDtypeStruct((BR, D2), jnp.float32),
        ],
    )(x, W_lh, b_lh2, alpha2, mask2)

    v_scaled = kvs[:, D:]

    out_vec = pl.pallas_call(
        _out_vec_body,
        grid=(D // O_BLK,),
        in_specs=[
            pl.BlockSpec((BR, D), lambda o: (0, 0)),
            pl.BlockSpec((O_BLK, D), lambda o: (o, 0)),
        ],
        out_specs=pl.BlockSpec((BR, O_BLK), lambda o: (0, o)),
        out_shape=jax.ShapeDtypeStruct((BR, D), jnp.float32),
    )(v_scaled, W_out)

    ov3 = out_vec.reshape(B, R, D)

    updated = pl.pallas_call(
        _assemble_body,
        grid=(B * NB,),
        in_specs=[
            pl.BlockSpec((1, N_BLK, D), lambda i: (i // NB, i % NB, 0)),
            pl.BlockSpec((1, N_BLK, R), lambda i: (i, 0, 0)),
            pl.BlockSpec((1, R, D), lambda i: (i // NB, 0, 0)),
        ],
        out_specs=pl.BlockSpec((1, N_BLK, D), lambda i: (i // NB, i % NB, 0)),
        out_shape=jax.ShapeDtypeStruct((B, N, D), jnp.float32),
    )(image_tokens, winner, ov3)

    layout_outputs = lo.reshape(B, R, D2)
    return updated, layout_outputs


# speedup vs baseline: 1.1272x; 1.1272x over previous
import functools

import jax
import jax.numpy as jnp
from jax import lax
from jax.experimental import pallas as pl
from jax.experimental.pallas import tpu as pltpu

B, R, N, L, D, H, Dh = 4, 8, 2048, 256, 3072, 24, 128
BR = B * R
D2 = 2 * D

E_BLK = 512
O_BLK = 512
N_BLK = 256
NB = N // N_BLK


def _layout_kv_body(x_ref, w_ref, b_ref, alpha_ref, mask_ref,
                    lo_ref, kvs_ref):
    x = x_ref[...]
    w = w_ref[...]
    kv = lax.dot_general(x, w, (((1,), (1,)), ((), ())),
                         preferred_element_type=jnp.float32)
    kv = kv + b_ref[...]
    valid = mask_ref[...] == 1.0
    lo_ref[...] = jnp.where(valid, kv, 0.0)
    kvs_ref[...] = kv * (1.0 - alpha_ref[...])


def _out_vec_body(v_ref, w_ref, o_ref):
    o_ref[...] = lax.dot_general(v_ref[...], w_ref[...],
                                 (((1,), (1,)), ((), ())),
                                 preferred_element_type=jnp.float32)


def _assemble_body(img_ref, idx_ref, mask_ref, ov_ref, out_ref):
    nb = pl.program_id(0) % NB
    n0 = nb * N_BLK
    ids = n0 + lax.broadcasted_iota(jnp.int32, (N_BLK, L), 0)
    winner = jnp.full((N_BLK, 1), -1, dtype=jnp.int32)
    for j in range(R):
        idx_j = idx_ref[0, j, :][None, :]
        hit = jnp.any(idx_j == ids, axis=1, keepdims=True)
        valid = mask_ref[0, j, 0] == 1.0
        winner = jnp.where(hit & valid, j, winner)
    onehot = (winner == lax.broadcasted_iota(jnp.int32, (N_BLK, R), 1))
    rows = lax.dot_general(onehot.astype(jnp.float32), ov_ref[0],
                           (((1,), (0,)), ((), ())),
                           preferred_element_type=jnp.float32)
    out_ref[0] = jnp.where(winner >= 0, rows, img_ref[0])


def kernel(instance_tokens, image_tokens, img_idxs, layout_masks, alpha,
           W_lh, b_lh, W_q, W_out):
    x = instance_tokens.reshape(BR, D)
    alpha2 = alpha.reshape(BR, 1)
    mask2 = layout_masks.reshape(BR, 1)
    b_lh2 = b_lh.reshape(1, D2)

    lo, kvs = pl.pallas_call(
        _layout_kv_body,
        grid=(D2 // E_BLK,),
        in_specs=[
            pl.BlockSpec((BR, D), lambda e: (0, 0)),
            pl.BlockSpec((E_BLK, D), lambda e: (e, 0)),
            pl.BlockSpec((1, E_BLK), lambda e: (0, e)),
            pl.BlockSpec((BR, 1), lambda e: (0, 0)),
            pl.BlockSpec((BR, 1), lambda e: (0, 0)),
        ],
        out_specs=[
            pl.BlockSpec((BR, E_BLK), lambda e: (0, e)),
            pl.BlockSpec((BR, E_BLK), lambda e: (0, e)),
        ],
        out_shape=[
            jax.ShapeDtypeStruct((BR, D2), jnp.float32),
            jax.ShapeDtypeStruct((BR, D2), jnp.float32),
        ],
    )(x, W_lh, b_lh2, alpha2, mask2)

    v_scaled = kvs[:, D:]

    out_vec = pl.pallas_call(
        _out_vec_body,
        grid=(D // O_BLK,),
        in_specs=[
            pl.BlockSpec((BR, D), lambda o: (0, 0)),
            pl.BlockSpec((O_BLK, D), lambda o: (o, 0)),
        ],
        out_specs=pl.BlockSpec((BR, O_BLK), lambda o: (0, o)),
        out_shape=jax.ShapeDtypeStruct((BR, D), jnp.float32),
    )(v_scaled, W_out)

    ov3 = out_vec.reshape(B, R, D)
    mask3 = layout_masks.reshape(B, R, 1)

    updated = pl.pallas_call(
        _assemble_body,
        grid=(B * NB,),
        in_specs=[
            pl.BlockSpec((1, N_BLK, D), lambda i: (i // NB, i % NB, 0)),
            pl.BlockSpec((1, R, L), lambda i: (i // NB, 0, 0)),
            pl.BlockSpec((1, R, 1), lambda i: (i // NB, 0, 0)),
            pl.BlockSpec((1, R, D), lambda i: (i // NB, 0, 0)),
        ],
        out_specs=pl.BlockSpec((1, N_BLK, D), lambda i: (i // NB, i % NB, 0)),
        out_shape=jax.ShapeDtypeStruct((B, N, D), jnp.float32),
    )(image_tokens, img_idxs, mask3, ov3)

    layout_outputs = lo.reshape(B, R, D2)
    return updated, layout_outputs
